# Initial kernel scaffold; baseline (speedup 1.0000x reference)
import jax
import jax.numpy as jnp
from jax.experimental import pallas as pl


def kernel(x, edge_index, batch, W1, b1, W2, b2, Wfc, bfc, Wreg, breg, Wcls, bcls):
    raise NotImplementedError("write your pallas kernel here")


# speedup vs baseline: 21.4451x; 21.4451x over previous
import functools

import jax
import jax.numpy as jnp
from jax import lax
from jax.experimental import pallas as pl
from jax.experimental.pallas import tpu as pltpu
from jax.experimental.pallas import tpu_sc as plsc

N = 10000
E = 160000
D = 256
H = 32
G = 64

NC = 2
NS = 16
NP = 10240
ROWS_PER_TILE = NP // NS
CH = 128
EP = 163840
EDGES_PER_TILE = EP // (NC * NS)
NCHUNK = EDGES_PER_TILE // CH
BN = 1024
GRID = NP // BN

_mesh = plsc.VectorSubcoreMesh(core_axis_name="c", subcore_axis_name="s")
_sc_params = pltpu.CompilerParams(use_tc_tiling_on_sc=False)


@functools.partial(
    pl.kernel,
    out_type=jax.ShapeDtypeStruct((NC, NP), jnp.float32),
    mesh=_mesh,
    scratch_types=[
        pltpu.VMEM((CH,), jnp.float32),
        pltpu.VMEM((NCHUNK, CH), jnp.int32),
        pltpu.VMEM_SHARED((NP,), jnp.float32),
    ],
    compiler_params=_sc_params,
)
def _deg_kernel(dst_hbm, out_hbm, ones_v, idx_v, acc_s):
    c = lax.axis_index("c")
    s = lax.axis_index("s")
    wid = c * NS + s
    z = jnp.zeros((16,), jnp.float32)
    for i in range(CH // 16):
        ones_v[pl.ds(i * 16, 16)] = z
    def _zero(j, _):
        pltpu.sync_copy(ones_v, acc_s.at[pl.ds(s * ROWS_PER_TILE + j * CH, CH)])
        return _
    lax.fori_loop(0, ROWS_PER_TILE // CH, _zero, None)
    o = jnp.ones((16,), jnp.float32)
    for i in range(CH // 16):
        ones_v[pl.ds(i * 16, 16)] = o
    pltpu.sync_copy(dst_hbm.at[pl.ds(wid * NCHUNK, NCHUNK)], idx_v)
    plsc.subcore_barrier()
    def _body(j, _):
        pltpu.sync_copy(ones_v, acc_s.at[idx_v.at[j]], add=True)
        return _
    lax.fori_loop(0, NCHUNK, _body, None)
    plsc.subcore_barrier()
    off = s * ROWS_PER_TILE
    pltpu.sync_copy(acc_s.at[pl.ds(off, ROWS_PER_TILE)],
                    out_hbm.at[c, pl.ds(off, ROWS_PER_TILE)])


@functools.partial(
    pl.kernel,
    out_type=jax.ShapeDtypeStruct((NC, NP, H), jnp.float32),
    mesh=_mesh,
    scratch_types=[
        pltpu.VMEM((NCHUNK, CH), jnp.int32),
        pltpu.VMEM((NCHUNK, CH), jnp.int32),
        pltpu.VMEM((CH, H), jnp.float32),
        pltpu.VMEM_SHARED((NP, H), jnp.float32),
        pltpu.SemaphoreType.DMA,
    ],
    compiler_params=_sc_params,
)
def _scatter_kernel(g_hbm, src_hbm, dst_hbm, out_hbm,
                    src_v, dst_v, rows_v, acc_s, sem):
    c = lax.axis_index("c")
    s = lax.axis_index("s")
    wid = c * NS + s
    z = jnp.zeros((16,), jnp.float32)
    def _zrow(j, _):
        rows_v[j, pl.ds(0, 16)] = z
        rows_v[j, pl.ds(16, 16)] = z
        return _
    lax.fori_loop(0, CH, _zrow, None)
    def _zero(j, _):
        pltpu.sync_copy(rows_v, acc_s.at[pl.ds(s * ROWS_PER_TILE + j * CH, CH)])
        return _
    lax.fori_loop(0, ROWS_PER_TILE // CH, _zero, None)
    pltpu.sync_copy(src_hbm.at[pl.ds(wid * NCHUNK, NCHUNK)], src_v)
    pltpu.sync_copy(dst_hbm.at[pl.ds(wid * NCHUNK, NCHUNK)], dst_v)
    plsc.subcore_barrier()
    def _body(j, _):
        pltpu.async_copy(g_hbm.at[src_v.at[j]], rows_v, sem).wait()
        pltpu.sync_copy(rows_v, acc_s.at[dst_v.at[j]], add=True)
        return _
    lax.fori_loop(0, NCHUNK, _body, None)
    plsc.subcore_barrier()
    off = s * ROWS_PER_TILE
    pltpu.sync_copy(acc_s.at[pl.ds(off, ROWS_PER_TILE)],
                    out_hbm.at[c, pl.ds(off, ROWS_PER_TILE)])


def _tc1_body(x_ref, w_ref, d0_ref, d1_ref, g_ref):
    dinv = lax.rsqrt(d0_ref[...] + d1_ref[...] + 1.0)
    h = jnp.dot(x_ref[...], w_ref[...], preferred_element_type=jnp.float32)
    g_ref[...] = h * dinv


def _tc1(xp, W1, d0, d1):
    return pl.pallas_call(
        _tc1_body,
        out_shape=jax.ShapeDtypeStruct((NP, H), jnp.float32),
        grid=(GRID,),
        in_specs=[
            pl.BlockSpec((BN, D), lambda i: (i, 0)),
            pl.BlockSpec((D, H), lambda i: (0, 0)),
            pl.BlockSpec((BN, 1), lambda i: (i, 0)),
            pl.BlockSpec((BN, 1), lambda i: (i, 0)),
        ],
        out_specs=pl.BlockSpec((BN, H), lambda i: (i, 0)),
    )(xp, W1, d0, d1)


def _tc2_body(p0_ref, p1_ref, g_ref, d0_ref, d1_ref, b_ref, w_ref, out_ref):
    dinv = lax.rsqrt(d0_ref[...] + d1_ref[...] + 1.0)
    pre = dinv * (p0_ref[...] + p1_ref[...] + g_ref[...]) + b_ref[...]
    h = jnp.maximum(pre, 0.0)
    out_ref[...] = jnp.dot(h, w_ref[...],
                           preferred_element_type=jnp.float32) * dinv


def _tc2(p0, p1, g1, d0, d1, b1, W2):
    return pl.pallas_call(
        _tc2_body,
        out_shape=jax.ShapeDtypeStruct((NP, H), jnp.float32),
        grid=(GRID,),
        in_specs=[
            pl.BlockSpec((BN, H), lambda i: (i, 0)),
            pl.BlockSpec((BN, H), lambda i: (i, 0)),
            pl.BlockSpec((BN, H), lambda i: (i, 0)),
            pl.BlockSpec((BN, 1), lambda i: (i, 0)),
            pl.BlockSpec((BN, 1), lambda i: (i, 0)),
            pl.BlockSpec((1, H), lambda i: (0, 0)),
            pl.BlockSpec((H, H), lambda i: (0, 0)),
        ],
        out_specs=pl.BlockSpec((BN, H), lambda i: (i, 0)),
    )(p0, p1, g1, d0, d1, b1, W2)


def _tc3_body(q0_ref, q1_ref, g_ref, d0_ref, d1_ref, b_ref, batch_ref,
              wfc_ref, bfc_ref, wreg_ref, breg_ref, wcls_ref, bcls_ref,
              reg_ref, cls_ref, sums_ref, cnt_ref):
    i = pl.program_id(0)

    @pl.when(i == 0)
    def _init():
        sums_ref[...] = jnp.zeros((G, H), jnp.float32)
        cnt_ref[...] = jnp.zeros((G, 1), jnp.float32)

    dinv = lax.rsqrt(d0_ref[...] + d1_ref[...] + 1.0)
    pre = dinv * (q0_ref[...] + q1_ref[...] + g_ref[...]) + b_ref[...]
    h = jnp.maximum(pre, 0.0)
    bvals = batch_ref[...]
    gids = lax.broadcasted_iota(jnp.int32, (G, BN), 0)
    onehot_t = (gids == bvals).astype(jnp.float32)
    sums_ref[...] += jnp.dot(onehot_t, h, preferred_element_type=jnp.float32)
    cnt_ref[...] += jnp.sum(onehot_t, axis=1, keepdims=True)

    @pl.when(i == pl.num_programs(0) - 1)
    def _final():
        pooled = sums_ref[...] / jnp.maximum(cnt_ref[...], 1.0)
        sfc = jnp.maximum(
            jnp.dot(pooled, wfc_ref[...],
                    preferred_element_type=jnp.float32) + bfc_ref[...], 0.0)
        reg_ref[...] = jnp.dot(sfc, wreg_ref[...],
                               preferred_element_type=jnp.float32) + breg_ref[...]
        cls_ref[...] = jnp.dot(sfc, wcls_ref[...],
                               preferred_element_type=jnp.float32) + bcls_ref[...]


def _tc3(q0, q1, g2, d0, d1, b2, batch2d, Wfc, bfc, Wreg, breg, Wcls, bcls):
    return pl.pallas_call(
        _tc3_body,
        out_shape=[jax.ShapeDtypeStruct((G, 2), jnp.float32),
                   jax.ShapeDtypeStruct((G, 2), jnp.float32)],
        grid=(GRID,),
        in_specs=[
            pl.BlockSpec((BN, H), lambda i: (i, 0)),
            pl.BlockSpec((BN, H), lambda i: (i, 0)),
            pl.BlockSpec((BN, H), lambda i: (i, 0)),
            pl.BlockSpec((BN, 1), lambda i: (i, 0)),
            pl.BlockSpec((BN, 1), lambda i: (i, 0)),
            pl.BlockSpec((1, H), lambda i: (0, 0)),
            pl.BlockSpec((1, BN), lambda i: (0, i)),
            pl.BlockSpec((H, H), lambda i: (0, 0)),
            pl.BlockSpec((1, H), lambda i: (0, 0)),
            pl.BlockSpec((H, 2), lambda i: (0, 0)),
            pl.BlockSpec((1, 2), lambda i: (0, 0)),
            pl.BlockSpec((H, 2), lambda i: (0, 0)),
            pl.BlockSpec((1, 2), lambda i: (0, 0)),
        ],
        out_specs=[pl.BlockSpec((G, 2), lambda i: (0, 0)),
                   pl.BlockSpec((G, 2), lambda i: (0, 0))],
        scratch_shapes=[pltpu.VMEM((G, H), jnp.float32),
                        pltpu.VMEM((G, 1), jnp.float32)],
    )(q0, q1, g2, d0, d1, b2, batch2d, Wfc, bfc, Wreg, breg, Wcls, bcls)


def kernel(x, edge_index, batch, W1, b1, W2, b2, Wfc, bfc, Wreg, breg,
           Wcls, bcls):
    xp = jnp.pad(x, ((0, NP - N), (0, 0)))
    pad_ids = (N + (jnp.arange(EP - E, dtype=jnp.int32) % (NP - N)))
    srcp = jnp.concatenate([edge_index[0], pad_ids]).reshape(EP // CH, CH)
    dstp = jnp.concatenate([edge_index[1], pad_ids]).reshape(EP // CH, CH)
    batchp = jnp.concatenate(
        [batch, jnp.full((NP - N,), G, jnp.int32)]).reshape(1, NP)

    deg = _deg_kernel(dstp)
    d0 = deg[0].reshape(NP, 1)
    d1 = deg[1].reshape(NP, 1)

    g1 = _tc1(xp, W1, d0, d1)
    p = _scatter_kernel(g1, srcp, dstp)
    g2 = _tc2(p[0], p[1], g1, d0, d1, b1.reshape(1, H), W2)
    q = _scatter_kernel(g2, srcp, dstp)
    reg, cls = _tc3(q[0], q[1], g2, d0, d1, b2.reshape(1, H), batchp,
                    Wfc, bfc.reshape(1, H), Wreg, breg.reshape(1, 2),
                    Wcls, bcls.reshape(1, 2))
    return (reg, cls)


# speedup vs baseline: 27.8105x; 1.2968x over previous
import functools

import jax
import jax.numpy as jnp
from jax import lax
from jax.experimental import pallas as pl
from jax.experimental.pallas import tpu as pltpu
from jax.experimental.pallas import tpu_sc as plsc

N = 10000
E = 160000
D = 256
H = 32
G = 64

NC = 2
NS = 16
NP = 10240
ROWS_PER_TILE = NP // NS
CH = 128
EP = 163840
EDGES_PER_TILE = EP // (NC * NS)
NCHUNK = EDGES_PER_TILE // CH
BN = 1024
GRID = NP // BN

_mesh = plsc.VectorSubcoreMesh(core_axis_name="c", subcore_axis_name="s")
_sc_params = pltpu.CompilerParams(use_tc_tiling_on_sc=False)


@functools.partial(
    pl.kernel,
    out_type=jax.ShapeDtypeStruct((NC, NP), jnp.float32),
    mesh=_mesh,
    scratch_types=[
        pltpu.VMEM((CH,), jnp.float32),
        pltpu.VMEM((NCHUNK, CH), jnp.int32),
        pltpu.VMEM_SHARED((NP,), jnp.float32),
    ],
    compiler_params=_sc_params,
)
def _deg_kernel(dst_hbm, out_hbm, ones_v, idx_v, acc_s):
    c = lax.axis_index("c")
    s = lax.axis_index("s")
    wid = c * NS + s
    z = jnp.zeros((16,), jnp.float32)
    for i in range(CH // 16):
        ones_v[pl.ds(i * 16, 16)] = z
    def _zero(j, _):
        pltpu.sync_copy(ones_v, acc_s.at[pl.ds(s * ROWS_PER_TILE + j * CH, CH)])
        return _
    lax.fori_loop(0, ROWS_PER_TILE // CH, _zero, None)
    o = jnp.ones((16,), jnp.float32)
    for i in range(CH // 16):
        ones_v[pl.ds(i * 16, 16)] = o
    pltpu.sync_copy(dst_hbm.at[pl.ds(wid * NCHUNK, NCHUNK)], idx_v)
    plsc.subcore_barrier()
    def _body(j, _):
        pltpu.sync_copy(ones_v, acc_s.at[idx_v.at[j]], add=True)
        return _
    lax.fori_loop(0, NCHUNK, _body, None)
    plsc.subcore_barrier()
    off = s * ROWS_PER_TILE
    pltpu.sync_copy(acc_s.at[pl.ds(off, ROWS_PER_TILE)],
                    out_hbm.at[c, pl.ds(off, ROWS_PER_TILE)])


@functools.partial(
    pl.kernel,
    out_type=jax.ShapeDtypeStruct((NC, NP, H), jnp.float32),
    mesh=_mesh,
    scratch_types=[
        pltpu.VMEM((NCHUNK, CH), jnp.int32),
        pltpu.VMEM((NCHUNK, CH), jnp.int32),
        pltpu.VMEM((CH, H), jnp.float32),
        pltpu.VMEM((CH, H), jnp.float32),
        pltpu.VMEM_SHARED((NP, H), jnp.float32),
        pltpu.SemaphoreType.DMA,
        pltpu.SemaphoreType.DMA,
    ],
    compiler_params=_sc_params,
)
def _scatter_kernel(g_hbm, src_hbm, dst_hbm, out_hbm,
                    src_v, dst_v, rows_a, rows_b, acc_s, sem_a, sem_b):
    c = lax.axis_index("c")
    s = lax.axis_index("s")
    wid = c * NS + s
    z = jnp.zeros((16,), jnp.float32)
    def _zrow(j, _):
        rows_a[j, pl.ds(0, 16)] = z
        rows_a[j, pl.ds(16, 16)] = z
        return _
    lax.fori_loop(0, CH, _zrow, None)
    def _zero(j, _):
        pltpu.sync_copy(rows_a, acc_s.at[pl.ds(s * ROWS_PER_TILE + j * CH, CH)])
        return _
    lax.fori_loop(0, ROWS_PER_TILE // CH, _zero, None)
    pltpu.sync_copy(src_hbm.at[pl.ds(wid * NCHUNK, NCHUNK)], src_v)
    pltpu.sync_copy(dst_hbm.at[pl.ds(wid * NCHUNK, NCHUNK)], dst_v)
    plsc.subcore_barrier()
    pltpu.async_copy(g_hbm.at[src_v.at[0]], rows_a, sem_a)
    pltpu.async_copy(g_hbm.at[src_v.at[1]], rows_b, sem_b)
    def _body(i, _):
        ja = 2 * i
        pltpu.make_async_copy(g_hbm.at[src_v.at[ja]], rows_a, sem_a).wait()
        pltpu.sync_copy(rows_a, acc_s.at[dst_v.at[ja]], add=True)
        @pl.when(ja + 2 < NCHUNK)
        def _na():
            pltpu.async_copy(g_hbm.at[src_v.at[ja + 2]], rows_a, sem_a)
        jb = 2 * i + 1
        pltpu.make_async_copy(g_hbm.at[src_v.at[jb]], rows_b, sem_b).wait()
        pltpu.sync_copy(rows_b, acc_s.at[dst_v.at[jb]], add=True)
        @pl.when(jb + 2 < NCHUNK)
        def _nb():
            pltpu.async_copy(g_hbm.at[src_v.at[jb + 2]], rows_b, sem_b)
        return _
    lax.fori_loop(0, NCHUNK // 2, _body, None)
    plsc.subcore_barrier()
    off = s * ROWS_PER_TILE
    pltpu.sync_copy(acc_s.at[pl.ds(off, ROWS_PER_TILE)],
                    out_hbm.at[c, pl.ds(off, ROWS_PER_TILE)])


def _tc1_body(x_ref, w_ref, d_ref, g_ref, dinv_ref):
    i = pl.program_id(0)
    dinv = lax.rsqrt(d_ref[0] + d_ref[1] + 1.0)
    rows = i * BN + lax.broadcasted_iota(jnp.int32, (BN, 1), 0)
    h = jnp.dot(x_ref[...], w_ref[...], preferred_element_type=jnp.float32)
    valid = rows < N
    g_ref[...] = jnp.where(valid, h * dinv, 0.0)
    dinv_ref[...] = jnp.broadcast_to(jnp.where(valid, dinv, 0.0), (BN, H))


def _tc1(x, W1, deg):
    return pl.pallas_call(
        _tc1_body,
        out_shape=[jax.ShapeDtypeStruct((NP, H), jnp.float32),
                   jax.ShapeDtypeStruct((NP, H), jnp.float32)],
        grid=(GRID,),
        in_specs=[
            pl.BlockSpec((BN, D), lambda i: (i, 0)),
            pl.BlockSpec((D, H), lambda i: (0, 0)),
            pl.BlockSpec((NC, BN, 1), lambda i: (0, i, 0)),
        ],
        out_specs=[pl.BlockSpec((BN, H), lambda i: (i, 0)),
                   pl.BlockSpec((BN, H), lambda i: (i, 0))],
    )(x, W1, deg)


def _tc2_body(p_ref, g_ref, dinv_ref, b_ref, w_ref, out_ref):
    dinv = dinv_ref[...]
    pre = dinv * (p_ref[0] + p_ref[1] + g_ref[...]) + b_ref[...]
    h = jnp.maximum(pre, 0.0)
    out_ref[...] = jnp.dot(h, w_ref[...],
                           preferred_element_type=jnp.float32) * dinv


def _tc2(p, g1, dinv32, b1, W2):
    return pl.pallas_call(
        _tc2_body,
        out_shape=jax.ShapeDtypeStruct((NP, H), jnp.float32),
        grid=(GRID,),
        in_specs=[
            pl.BlockSpec((NC, BN, H), lambda i: (0, i, 0)),
            pl.BlockSpec((BN, H), lambda i: (i, 0)),
            pl.BlockSpec((BN, H), lambda i: (i, 0)),
            pl.BlockSpec((1, H), lambda i: (0, 0)),
            pl.BlockSpec((H, H), lambda i: (0, 0)),
        ],
        out_specs=pl.BlockSpec((BN, H), lambda i: (i, 0)),
    )(p, g1, dinv32, b1, W2)


def _tc3_body(q_ref, g_ref, dinv_ref, b_ref, batch_ref,
              wfc_ref, bfc_ref, wreg_ref, breg_ref, wcls_ref, bcls_ref,
              reg_ref, cls_ref, sums_ref, cnt_ref):
    i = pl.program_id(0)

    @pl.when(i == 0)
    def _init():
        sums_ref[...] = jnp.zeros((G, H), jnp.float32)
        cnt_ref[...] = jnp.zeros((G, 1), jnp.float32)

    pre = dinv_ref[...] * (q_ref[0] + q_ref[1] + g_ref[...]) + b_ref[...]
    h = jnp.maximum(pre, 0.0)
    bvals = batch_ref[...]
    gids = lax.broadcasted_iota(jnp.int32, (G, BN), 0)
    onehot_t = (gids == bvals).astype(jnp.float32)
    sums_ref[...] += jnp.dot(onehot_t, h, preferred_element_type=jnp.float32)
    cnt_ref[...] += jnp.sum(onehot_t, axis=1, keepdims=True)

    @pl.when(i == pl.num_programs(0) - 1)
    def _final():
        pooled = sums_ref[...] / jnp.maximum(cnt_ref[...], 1.0)
        sfc = jnp.maximum(
            jnp.dot(pooled, wfc_ref[...],
                    preferred_element_type=jnp.float32) + bfc_ref[...], 0.0)
        reg_ref[...] = jnp.dot(sfc, wreg_ref[...],
                               preferred_element_type=jnp.float32) + breg_ref[...]
        cls_ref[...] = jnp.dot(sfc, wcls_ref[...],
                               preferred_element_type=jnp.float32) + bcls_ref[...]


def _tc3(q, g2, dinv32, b2, batch2d, Wfc, bfc, Wreg, breg, Wcls, bcls):
    return pl.pallas_call(
        _tc3_body,
        out_shape=[jax.ShapeDtypeStruct((G, 2), jnp.float32),
                   jax.ShapeDtypeStruct((G, 2), jnp.float32)],
        grid=(GRID,),
        in_specs=[
            pl.BlockSpec((NC, BN, H), lambda i: (0, i, 0)),
            pl.BlockSpec((BN, H), lambda i: (i, 0)),
            pl.BlockSpec((BN, H), lambda i: (i, 0)),
            pl.BlockSpec((1, H), lambda i: (0, 0)),
            pl.BlockSpec((1, BN), lambda i: (0, i)),
            pl.BlockSpec((H, H), lambda i: (0, 0)),
            pl.BlockSpec((1, H), lambda i: (0, 0)),
            pl.BlockSpec((H, 2), lambda i: (0, 0)),
            pl.BlockSpec((1, 2), lambda i: (0, 0)),
            pl.BlockSpec((H, 2), lambda i: (0, 0)),
            pl.BlockSpec((1, 2), lambda i: (0, 0)),
        ],
        out_specs=[pl.BlockSpec((G, 2), lambda i: (0, 0)),
                   pl.BlockSpec((G, 2), lambda i: (0, 0))],
        scratch_shapes=[pltpu.VMEM((G, H), jnp.float32),
                        pltpu.VMEM((G, 1), jnp.float32)],
    )(q, g2, dinv32, b2, batch2d, Wfc, bfc, Wreg, breg, Wcls, bcls)


def kernel(x, edge_index, batch, W1, b1, W2, b2, Wfc, bfc, Wreg, breg,
           Wcls, bcls):
    pad_ids = (N + (jnp.arange(EP - E, dtype=jnp.int32) % (NP - N)))
    srcp = jnp.concatenate([edge_index[0], pad_ids]).reshape(EP // CH, CH)
    dstp = jnp.concatenate([edge_index[1], pad_ids]).reshape(EP // CH, CH)
    batchp = jnp.concatenate(
        [batch, jnp.full((NP - N,), G, jnp.int32)]).reshape(1, NP)

    deg = _deg_kernel(dstp).reshape(NC, NP, 1)
    g1, dinv32 = _tc1(x, W1, deg)
    p = _scatter_kernel(g1, srcp, dstp)
    g2 = _tc2(p, g1, dinv32, b1.reshape(1, H), W2)
    q = _scatter_kernel(g2, srcp, dstp)
    reg, cls = _tc3(q, g2, dinv32, b2.reshape(1, H), batchp,
                    Wfc, bfc.reshape(1, H), Wreg, breg.reshape(1, 2),
                    Wcls, bcls.reshape(1, 2))
    return (reg, cls)


# speedup vs baseline: 32.5164x; 1.1692x over previous
import functools

import jax
import jax.numpy as jnp
from jax import lax
from jax.experimental import pallas as pl
from jax.experimental.pallas import tpu as pltpu
from jax.experimental.pallas import tpu_sc as plsc

N = 10000
E = 160000
D = 256
H = 32
G = 64

NC = 2
NS = 16
NP = 10240
ROWS_PER_TILE = NP // NS
CH = 128
EP = 163840
EDGES_PER_TILE = EP // (NC * NS)
NCHUNK = EDGES_PER_TILE // CH
NB = 4
BN = 2048
BR = BN // 128
GRID = NP // BN

_mesh = plsc.VectorSubcoreMesh(core_axis_name="c", subcore_axis_name="s")
_sc_params = pltpu.CompilerParams(use_tc_tiling_on_sc=False)


@functools.partial(
    pl.kernel,
    out_type=jax.ShapeDtypeStruct((NC, NP // 128, 128), jnp.float32),
    mesh=_mesh,
    scratch_types=[
        pltpu.VMEM((CH,), jnp.float32),
        pltpu.VMEM((NCHUNK, CH), jnp.int32),
        pltpu.VMEM_SHARED((NP,), jnp.float32),
    ],
    compiler_params=_sc_params,
)
def _deg_kernel(dst_hbm, out_hbm, ones_v, idx_v, acc_s):
    c = lax.axis_index("c")
    s = lax.axis_index("s")
    wid = c * NS + s
    z = jnp.zeros((16,), jnp.float32)
    for i in range(CH // 16):
        ones_v[pl.ds(i * 16, 16)] = z
    def _zero(j, _):
        pltpu.sync_copy(ones_v, acc_s.at[pl.ds(s * ROWS_PER_TILE + j * CH, CH)])
        return _
    lax.fori_loop(0, ROWS_PER_TILE // CH, _zero, None)
    o = jnp.ones((16,), jnp.float32)
    for i in range(CH // 16):
        ones_v[pl.ds(i * 16, 16)] = o
    pltpu.sync_copy(dst_hbm.at[pl.ds(wid * NCHUNK, NCHUNK)], idx_v)
    plsc.subcore_barrier()
    def _body(j, _):
        pltpu.sync_copy(ones_v, acc_s.at[idx_v.at[j]], add=True)
        return _
    lax.fori_loop(0, NCHUNK, _body, None)
    plsc.subcore_barrier()
    def _out(k, _):
        pltpu.sync_copy(acc_s.at[pl.ds(s * ROWS_PER_TILE + k * 128, 128)],
                        out_hbm.at[c, s * (ROWS_PER_TILE // 128) + k])
        return _
    lax.fori_loop(0, ROWS_PER_TILE // 128, _out, None)


@functools.partial(
    pl.kernel,
    out_type=jax.ShapeDtypeStruct((NC, NP, H), jnp.float32),
    mesh=_mesh,
    scratch_types=(
        [pltpu.VMEM((NCHUNK, CH), jnp.int32)] * 2
        + [pltpu.VMEM((CH, H), jnp.float32)] * NB
        + [pltpu.VMEM_SHARED((NP, H), jnp.float32)]
        + [pltpu.SemaphoreType.DMA] * (2 * NB)
    ),
    compiler_params=_sc_params,
)
def _scatter_kernel(g_hbm, src_hbm, dst_hbm, out_hbm, src_v, dst_v, *rest):
    rows = list(rest[:NB])
    acc_s = rest[NB]
    sem_g = list(rest[NB + 1:NB + 1 + NB])
    sem_s = list(rest[NB + 1 + NB:])
    c = lax.axis_index("c")
    s = lax.axis_index("s")
    wid = c * NS + s
    z = jnp.zeros((16,), jnp.float32)
    def _zrow(j, _):
        rows[0][j, pl.ds(0, 16)] = z
        rows[0][j, pl.ds(16, 16)] = z
        return _
    lax.fori_loop(0, CH, _zrow, None)
    def _zero(j, _):
        pltpu.sync_copy(rows[0], acc_s.at[pl.ds(s * ROWS_PER_TILE + j * CH, CH)])
        return _
    lax.fori_loop(0, ROWS_PER_TILE // CH, _zero, None)
    pltpu.sync_copy(src_hbm.at[pl.ds(wid * NCHUNK, NCHUNK)], src_v)
    pltpu.sync_copy(dst_hbm.at[pl.ds(wid * NCHUNK, NCHUNK)], dst_v)
    plsc.subcore_barrier()
    pltpu.async_copy(g_hbm.at[src_v.at[0]], rows[0], sem_g[0])
    pltpu.async_copy(g_hbm.at[src_v.at[1]], rows[1], sem_g[1])
    def _body(i, _):
        for b in range(NB):
            j = NB * i + b
            pltpu.make_async_copy(g_hbm.at[src_v.at[j]], rows[b], sem_g[b]).wait()
            pltpu.async_copy(rows[b], acc_s.at[dst_v.at[j]], sem_s[b], add=True)
            b2 = (b + 2) % NB
            @pl.when(j + 2 < NCHUNK)
            def _issue():
                @pl.when(j >= 2)
                def _drain():
                    pltpu.make_async_copy(
                        rows[b2], acc_s.at[dst_v.at[j]], sem_s[b2]).wait()
                pltpu.async_copy(g_hbm.at[src_v.at[j + 2]], rows[b2], sem_g[b2])
        return _
    lax.fori_loop(0, NCHUNK // NB, _body, None)
    for b in range(NB):
        pltpu.make_async_copy(rows[b], acc_s.at[dst_v.at[0]], sem_s[b]).wait()
    plsc.subcore_barrier()
    off = s * ROWS_PER_TILE
    pltpu.sync_copy(acc_s.at[pl.ds(off, ROWS_PER_TILE)],
                    out_hbm.at[c, pl.ds(off, ROWS_PER_TILE)])


def _expand_rows(col_view):
    a_rows = lax.broadcasted_iota(jnp.int32, (BN, BR), 0) // 128
    a_cols = lax.broadcasted_iota(jnp.int32, (BN, BR), 1)
    sel = (a_rows == a_cols).astype(jnp.float32)
    o1 = jnp.dot(sel, col_view, preferred_element_type=jnp.float32)
    m_rows = lax.broadcasted_iota(jnp.int32, (BN, 128), 0) % 128
    m_cols = lax.broadcasted_iota(jnp.int32, (BN, 128), 1)
    msk = (m_rows == m_cols).astype(jnp.float32)
    return jnp.sum(o1 * msk, axis=1, keepdims=True)


def _tc1_body(x_ref, w_ref, d_ref, g_ref, dinv_ref):
    i = pl.program_id(0)
    dview = lax.rsqrt(d_ref[0] + d_ref[1] + 1.0)
    dinv = _expand_rows(dview)
    rows = i * BN + lax.broadcasted_iota(jnp.int32, (BN, 1), 0)
    h = jnp.dot(x_ref[...], w_ref[...], preferred_element_type=jnp.float32)
    valid = rows < N
    g_ref[...] = jnp.where(valid, h * dinv, 0.0)
    dinv_ref[...] = jnp.broadcast_to(jnp.where(valid, dinv, 0.0), (BN, H))


def _tc1(x, W1, deg):
    return pl.pallas_call(
        _tc1_body,
        out_shape=[jax.ShapeDtypeStruct((NP, H), jnp.float32),
                   jax.ShapeDtypeStruct((NP, H), jnp.float32)],
        grid=(GRID,),
        in_specs=[
            pl.BlockSpec((BN, D), lambda i: (i, 0)),
            pl.BlockSpec((D, H), lambda i: (0, 0)),
            pl.BlockSpec((NC, BR, 128), lambda i: (0, i, 0)),
        ],
        out_specs=[pl.BlockSpec((BN, H), lambda i: (i, 0)),
                   pl.BlockSpec((BN, H), lambda i: (i, 0))],
    )(x, W1, deg)


def _tc2_body(p_ref, g_ref, dinv_ref, b_ref, w_ref, out_ref):
    dinv = dinv_ref[...]
    pre = dinv * (p_ref[0] + p_ref[1] + g_ref[...]) + b_ref[...]
    h = jnp.maximum(pre, 0.0)
    out_ref[...] = jnp.dot(h, w_ref[...],
                           preferred_element_type=jnp.float32) * dinv


def _tc2(p, g1, dinv32, b1, W2):
    return pl.pallas_call(
        _tc2_body,
        out_shape=jax.ShapeDtypeStruct((NP, H), jnp.float32),
        grid=(GRID,),
        in_specs=[
            pl.BlockSpec((NC, BN, H), lambda i: (0, i, 0)),
            pl.BlockSpec((BN, H), lambda i: (i, 0)),
            pl.BlockSpec((BN, H), lambda i: (i, 0)),
            pl.BlockSpec((1, H), lambda i: (0, 0)),
            pl.BlockSpec((H, H), lambda i: (0, 0)),
        ],
        out_specs=pl.BlockSpec((BN, H), lambda i: (i, 0)),
    )(p, g1, dinv32, b1, W2)


def _tc3_body(q_ref, g_ref, dinv_ref, b_ref, batch_ref,
              wfc_ref, bfc_ref, wreg_ref, breg_ref, wcls_ref, bcls_ref,
              reg_ref, cls_ref, sums_ref, cnt_ref):
    i = pl.program_id(0)

    @pl.when(i == 0)
    def _init():
        sums_ref[...] = jnp.zeros((G, H), jnp.float32)
        cnt_ref[...] = jnp.zeros((G, 1), jnp.float32)

    pre = dinv_ref[...] * (q_ref[0] + q_ref[1] + g_ref[...]) + b_ref[...]
    h = jnp.maximum(pre, 0.0)
    bvals = batch_ref[...]
    gids = lax.broadcasted_iota(jnp.int32, (G, BN), 0)
    onehot_t = (gids == bvals).astype(jnp.float32)
    sums_ref[...] += jnp.dot(onehot_t, h, preferred_element_type=jnp.float32)
    cnt_ref[...] += jnp.sum(onehot_t, axis=1, keepdims=True)

    @pl.when(i == pl.num_programs(0) - 1)
    def _final():
        pooled = sums_ref[...] / jnp.maximum(cnt_ref[...], 1.0)
        sfc = jnp.maximum(
            jnp.dot(pooled, wfc_ref[...],
                    preferred_element_type=jnp.float32) + bfc_ref[...], 0.0)
        reg_ref[...] = jnp.dot(sfc, wreg_ref[...],
                               preferred_element_type=jnp.float32) + breg_ref[...]
        cls_ref[...] = jnp.dot(sfc, wcls_ref[...],
                               preferred_element_type=jnp.float32) + bcls_ref[...]


def _tc3(q, g2, dinv32, b2, batch2d, Wfc, bfc, Wreg, breg, Wcls, bcls):
    return pl.pallas_call(
        _tc3_body,
        out_shape=[jax.ShapeDtypeStruct((G, 2), jnp.float32),
                   jax.ShapeDtypeStruct((G, 2), jnp.float32)],
        grid=(GRID,),
        in_specs=[
            pl.BlockSpec((NC, BN, H), lambda i: (0, i, 0)),
            pl.BlockSpec((BN, H), lambda i: (i, 0)),
            pl.BlockSpec((BN, H), lambda i: (i, 0)),
            pl.BlockSpec((1, H), lambda i: (0, 0)),
            pl.BlockSpec((1, BN), lambda i: (0, i)),
            pl.BlockSpec((H, H), lambda i: (0, 0)),
            pl.BlockSpec((1, H), lambda i: (0, 0)),
            pl.BlockSpec((H, 2), lambda i: (0, 0)),
            pl.BlockSpec((1, 2), lambda i: (0, 0)),
            pl.BlockSpec((H, 2), lambda i: (0, 0)),
            pl.BlockSpec((1, 2), lambda i: (0, 0)),
        ],
        out_specs=[pl.BlockSpec((G, 2), lambda i: (0, 0)),
                   pl.BlockSpec((G, 2), lambda i: (0, 0))],
        scratch_shapes=[pltpu.VMEM((G, H), jnp.float32),
                        pltpu.VMEM((G, 1), jnp.float32)],
    )(q, g2, dinv32, b2, batch2d, Wfc, bfc, Wreg, breg, Wcls, bcls)


def kernel(x, edge_index, batch, W1, b1, W2, b2, Wfc, bfc, Wreg, breg,
           Wcls, bcls):
    pad_ids = (N + (jnp.arange(EP - E, dtype=jnp.int32) % (NP - N)))
    srcp = jnp.concatenate([edge_index[0], pad_ids]).reshape(EP // CH, CH)
    dstp = jnp.concatenate([edge_index[1], pad_ids]).reshape(EP // CH, CH)
    batchp = jnp.concatenate(
        [batch, jnp.full((NP - N,), G, jnp.int32)]).reshape(1, NP)

    deg = _deg_kernel(dstp)
    g1, dinv32 = _tc1(x, W1, deg)
    p = _scatter_kernel(g1, srcp, dstp)
    g2 = _tc2(p, g1, dinv32, b1.reshape(1, H), W2)
    q = _scatter_kernel(g2, srcp, dstp)
    reg, cls = _tc3(q, g2, dinv32, b2.reshape(1, H), batchp,
                    Wfc, bfc.reshape(1, H), Wreg, breg.reshape(1, 2),
                    Wcls, bcls.reshape(1, 2))
    return (reg, cls)


# speedup vs baseline: 33.8864x; 1.0421x over previous
import functools

import jax
import jax.numpy as jnp
from jax import lax
from jax.experimental import pallas as pl
from jax.experimental.pallas import tpu as pltpu
from jax.experimental.pallas import tpu_sc as plsc

N = 10000
E = 160000
D = 256
H = 32
G = 64

NC = 2
NS = 16
NP = 10240
ROWS_PER_TILE = NP // NS
CH = 128
EP = 163840
EDGES_PER_TILE = EP // (NC * NS)
NCHUNK = EDGES_PER_TILE // CH
NB = 4
BN = 2048
BR = BN // 128
GRID = NP // BN

_mesh = plsc.VectorSubcoreMesh(core_axis_name="c", subcore_axis_name="s")
_sc_params = pltpu.CompilerParams(use_tc_tiling_on_sc=False)


@functools.partial(
    pl.kernel,
    out_type=jax.ShapeDtypeStruct((NC, NP // 128, 128), jnp.float32),
    mesh=_mesh,
    scratch_types=[
        pltpu.VMEM((CH,), jnp.float32),
        pltpu.VMEM((NCHUNK, CH), jnp.int32),
        pltpu.VMEM_SHARED((NP,), jnp.float32),
    ],
    compiler_params=_sc_params,
)
def _deg_kernel(dst_hbm, out_hbm, ones_v, idx_v, acc_s):
    c = lax.axis_index("c")
    s = lax.axis_index("s")
    wid = c * NS + s
    z = jnp.zeros((16,), jnp.float32)
    for i in range(CH // 16):
        ones_v[pl.ds(i * 16, 16)] = z
    def _zero(j, _):
        pltpu.sync_copy(ones_v, acc_s.at[pl.ds(s * ROWS_PER_TILE + j * CH, CH)])
        return _
    lax.fori_loop(0, ROWS_PER_TILE // CH, _zero, None)
    o = jnp.ones((16,), jnp.float32)
    for i in range(CH // 16):
        ones_v[pl.ds(i * 16, 16)] = o
    pltpu.sync_copy(dst_hbm.at[pl.ds(wid * NCHUNK, NCHUNK)], idx_v)
    plsc.subcore_barrier()
    def _body(j, _):
        pltpu.sync_copy(ones_v, acc_s.at[idx_v.at[j]], add=True)
        return _
    lax.fori_loop(0, NCHUNK, _body, None)
    plsc.subcore_barrier()
    def _out(k, _):
        pltpu.sync_copy(acc_s.at[pl.ds(s * ROWS_PER_TILE + k * 128, 128)],
                        out_hbm.at[c, s * (ROWS_PER_TILE // 128) + k])
        return _
    lax.fori_loop(0, ROWS_PER_TILE // 128, _out, None)


@functools.partial(
    pl.kernel,
    out_type=jax.ShapeDtypeStruct((NC, NP, H), jnp.float32),
    mesh=_mesh,
    scratch_types=(
        [pltpu.VMEM((NCHUNK, CH), jnp.int32)] * 2
        + [pltpu.VMEM((CH, H), jnp.float32)] * NB
        + [pltpu.VMEM_SHARED((NP, H), jnp.float32)]
        + [pltpu.VMEM_SHARED((NP, H), jnp.float32)]
        + [pltpu.SemaphoreType.DMA] * (2 * NB)
    ),
    compiler_params=_sc_params,
)
def _scatter_kernel(g_hbm, src_hbm, dst_hbm, out_hbm, src_v, dst_v, *rest):
    rows = list(rest[:NB])
    acc_s = rest[NB]
    g_s = rest[NB + 1]
    sem_g = list(rest[NB + 2:NB + 2 + NB])
    sem_s = list(rest[NB + 2 + NB:])
    c = lax.axis_index("c")
    s = lax.axis_index("s")
    wid = c * NS + s
    z = jnp.zeros((16,), jnp.float32)
    def _zrow(j, _):
        rows[0][j, pl.ds(0, 16)] = z
        rows[0][j, pl.ds(16, 16)] = z
        return _
    lax.fori_loop(0, CH, _zrow, None)
    def _zero(j, _):
        pltpu.sync_copy(rows[0], acc_s.at[pl.ds(s * ROWS_PER_TILE + j * CH, CH)])
        return _
    lax.fori_loop(0, ROWS_PER_TILE // CH, _zero, None)
    pltpu.sync_copy(src_hbm.at[pl.ds(wid * NCHUNK, NCHUNK)], src_v)
    pltpu.sync_copy(dst_hbm.at[pl.ds(wid * NCHUNK, NCHUNK)], dst_v)
    off = s * ROWS_PER_TILE
    pltpu.sync_copy(g_hbm.at[pl.ds(off, ROWS_PER_TILE)],
                    g_s.at[pl.ds(off, ROWS_PER_TILE)])
    plsc.subcore_barrier()
    pltpu.async_copy(g_s.at[src_v.at[0]], rows[0], sem_g[0])
    pltpu.async_copy(g_s.at[src_v.at[1]], rows[1], sem_g[1])
    def _body(i, _):
        for b in range(NB):
            j = NB * i + b
            pltpu.make_async_copy(g_s.at[src_v.at[j]], rows[b], sem_g[b]).wait()
            pltpu.async_copy(rows[b], acc_s.at[dst_v.at[j]], sem_s[b], add=True)
            b2 = (b + 2) % NB
            @pl.when(j + 2 < NCHUNK)
            def _issue():
                @pl.when(j >= 2)
                def _drain():
                    pltpu.make_async_copy(
                        rows[b2], acc_s.at[dst_v.at[j]], sem_s[b2]).wait()
                pltpu.async_copy(g_s.at[src_v.at[j + 2]], rows[b2], sem_g[b2])
        return _
    lax.fori_loop(0, NCHUNK // NB, _body, None)
    for b in range(NB):
        pltpu.make_async_copy(rows[b], acc_s.at[dst_v.at[0]], sem_s[b]).wait()
    plsc.subcore_barrier()
    off = s * ROWS_PER_TILE
    pltpu.sync_copy(acc_s.at[pl.ds(off, ROWS_PER_TILE)],
                    out_hbm.at[c, pl.ds(off, ROWS_PER_TILE)])


def _expand_rows(col_view):
    a_rows = lax.broadcasted_iota(jnp.int32, (BN, BR), 0) // 128
    a_cols = lax.broadcasted_iota(jnp.int32, (BN, BR), 1)
    sel = (a_rows == a_cols).astype(jnp.float32)
    o1 = jnp.dot(sel, col_view, preferred_element_type=jnp.float32)
    m_rows = lax.broadcasted_iota(jnp.int32, (BN, 128), 0) % 128
    m_cols = lax.broadcasted_iota(jnp.int32, (BN, 128), 1)
    msk = (m_rows == m_cols).astype(jnp.float32)
    return jnp.sum(o1 * msk, axis=1, keepdims=True)


def _tc1_body(x_ref, w_ref, d_ref, g_ref, dinv_ref):
    i = pl.program_id(0)
    dview = lax.rsqrt(d_ref[0] + d_ref[1] + 1.0)
    dinv = _expand_rows(dview)
    rows = i * BN + lax.broadcasted_iota(jnp.int32, (BN, 1), 0)
    h = jnp.dot(x_ref[...], w_ref[...], preferred_element_type=jnp.float32)
    valid = rows < N
    g_ref[...] = jnp.where(valid, h * dinv, 0.0)
    dinv_ref[...] = jnp.broadcast_to(jnp.where(valid, dinv, 0.0), (BN, H))


def _tc1(x, W1, deg):
    return pl.pallas_call(
        _tc1_body,
        out_shape=[jax.ShapeDtypeStruct((NP, H), jnp.float32),
                   jax.ShapeDtypeStruct((NP, H), jnp.float32)],
        grid=(GRID,),
        in_specs=[
            pl.BlockSpec((BN, D), lambda i: (i, 0)),
            pl.BlockSpec((D, H), lambda i: (0, 0)),
            pl.BlockSpec((NC, BR, 128), lambda i: (0, i, 0)),
        ],
        out_specs=[pl.BlockSpec((BN, H), lambda i: (i, 0)),
                   pl.BlockSpec((BN, H), lambda i: (i, 0))],
    )(x, W1, deg)


def _tc2_body(p_ref, g_ref, dinv_ref, b_ref, w_ref, out_ref):
    dinv = dinv_ref[...]
    pre = dinv * (p_ref[0] + p_ref[1] + g_ref[...]) + b_ref[...]
    h = jnp.maximum(pre, 0.0)
    out_ref[...] = jnp.dot(h, w_ref[...],
                           preferred_element_type=jnp.float32) * dinv


def _tc2(p, g1, dinv32, b1, W2):
    return pl.pallas_call(
        _tc2_body,
        out_shape=jax.ShapeDtypeStruct((NP, H), jnp.float32),
        grid=(GRID,),
        in_specs=[
            pl.BlockSpec((NC, BN, H), lambda i: (0, i, 0)),
            pl.BlockSpec((BN, H), lambda i: (i, 0)),
            pl.BlockSpec((BN, H), lambda i: (i, 0)),
            pl.BlockSpec((1, H), lambda i: (0, 0)),
            pl.BlockSpec((H, H), lambda i: (0, 0)),
        ],
        out_specs=pl.BlockSpec((BN, H), lambda i: (i, 0)),
    )(p, g1, dinv32, b1, W2)


def _tc3_body(q_ref, g_ref, dinv_ref, b_ref, batch_ref,
              wfc_ref, bfc_ref, wreg_ref, breg_ref, wcls_ref, bcls_ref,
              reg_ref, cls_ref, sums_ref, cnt_ref):
    i = pl.program_id(0)

    @pl.when(i == 0)
    def _init():
        sums_ref[...] = jnp.zeros((G, H), jnp.float32)
        cnt_ref[...] = jnp.zeros((G, 1), jnp.float32)

    pre = dinv_ref[...] * (q_ref[0] + q_ref[1] + g_ref[...]) + b_ref[...]
    h = jnp.maximum(pre, 0.0)
    bvals = batch_ref[...]
    gids = lax.broadcasted_iota(jnp.int32, (G, BN), 0)
    onehot_t = (gids == bvals).astype(jnp.float32)
    sums_ref[...] += jnp.dot(onehot_t, h, preferred_element_type=jnp.float32)
    cnt_ref[...] += jnp.sum(onehot_t, axis=1, keepdims=True)

    @pl.when(i == pl.num_programs(0) - 1)
    def _final():
        pooled = sums_ref[...] / jnp.maximum(cnt_ref[...], 1.0)
        sfc = jnp.maximum(
            jnp.dot(pooled, wfc_ref[...],
                    preferred_element_type=jnp.float32) + bfc_ref[...], 0.0)
        reg_ref[...] = jnp.dot(sfc, wreg_ref[...],
                               preferred_element_type=jnp.float32) + breg_ref[...]
        cls_ref[...] = jnp.dot(sfc, wcls_ref[...],
                               preferred_element_type=jnp.float32) + bcls_ref[...]


def _tc3(q, g2, dinv32, b2, batch2d, Wfc, bfc, Wreg, breg, Wcls, bcls):
    return pl.pallas_call(
        _tc3_body,
        out_shape=[jax.ShapeDtypeStruct((G, 2), jnp.float32),
                   jax.ShapeDtypeStruct((G, 2), jnp.float32)],
        grid=(GRID,),
        in_specs=[
            pl.BlockSpec((NC, BN, H), lambda i: (0, i, 0)),
            pl.BlockSpec((BN, H), lambda i: (i, 0)),
            pl.BlockSpec((BN, H), lambda i: (i, 0)),
            pl.BlockSpec((1, H), lambda i: (0, 0)),
            pl.BlockSpec((1, BN), lambda i: (0, i)),
            pl.BlockSpec((H, H), lambda i: (0, 0)),
            pl.BlockSpec((1, H), lambda i: (0, 0)),
            pl.BlockSpec((H, 2), lambda i: (0, 0)),
            pl.BlockSpec((1, 2), lambda i: (0, 0)),
            pl.BlockSpec((H, 2), lambda i: (0, 0)),
            pl.BlockSpec((1, 2), lambda i: (0, 0)),
        ],
        out_specs=[pl.BlockSpec((G, 2), lambda i: (0, 0)),
                   pl.BlockSpec((G, 2), lambda i: (0, 0))],
        scratch_shapes=[pltpu.VMEM((G, H), jnp.float32),
                        pltpu.VMEM((G, 1), jnp.float32)],
    )(q, g2, dinv32, b2, batch2d, Wfc, bfc, Wreg, breg, Wcls, bcls)


def kernel(x, edge_index, batch, W1, b1, W2, b2, Wfc, bfc, Wreg, breg,
           Wcls, bcls):
    pad_ids = (N + (jnp.arange(EP - E, dtype=jnp.int32) % (NP - N)))
    srcp = jnp.concatenate([edge_index[0], pad_ids]).reshape(EP // CH, CH)
    dstp = jnp.concatenate([edge_index[1], pad_ids]).reshape(EP // CH, CH)
    batchp = jnp.concatenate(
        [batch, jnp.full((NP - N,), G, jnp.int32)]).reshape(1, NP)

    deg = _deg_kernel(dstp)
    g1, dinv32 = _tc1(x, W1, deg)
    p = _scatter_kernel(g1, srcp, dstp)
    g2 = _tc2(p, g1, dinv32, b1.reshape(1, H), W2)
    q = _scatter_kernel(g2, srcp, dstp)
    reg, cls = _tc3(q, g2, dinv32, b2.reshape(1, H), batchp,
                    Wfc, bfc.reshape(1, H), Wreg, breg.reshape(1, 2),
                    Wcls, bcls.reshape(1, 2))
    return (reg, cls)


# speedup vs baseline: 35.3007x; 1.0417x over previous
import functools

import jax
import jax.numpy as jnp
from jax import lax
from jax.experimental import pallas as pl
from jax.experimental.pallas import tpu as pltpu
from jax.experimental.pallas import tpu_sc as plsc

N = 10000
E = 160000
D = 256
H = 32
G = 64

NC = 2
NS = 16
NP = 10240
ROWS_PER_TILE = NP // NS
CH = 128
EP = 163840
EDGES_PER_TILE = EP // (NC * NS)
NCHUNK = EDGES_PER_TILE // CH
NB = 8
IG = 4
BN = 2048
BR = BN // 128
GRID = NP // BN

_mesh = plsc.VectorSubcoreMesh(core_axis_name="c", subcore_axis_name="s")
_sc_params = pltpu.CompilerParams(use_tc_tiling_on_sc=False)


@functools.partial(
    pl.kernel,
    out_type=jax.ShapeDtypeStruct((NC, NP // 128, 128), jnp.float32),
    mesh=_mesh,
    scratch_types=[
        pltpu.VMEM((CH,), jnp.float32),
        pltpu.VMEM((NCHUNK, CH), jnp.int32),
        pltpu.VMEM_SHARED((NP,), jnp.float32),
    ],
    compiler_params=_sc_params,
)
def _deg_kernel(dst_hbm, out_hbm, ones_v, idx_v, acc_s):
    c = lax.axis_index("c")
    s = lax.axis_index("s")
    wid = c * NS + s
    z = jnp.zeros((16,), jnp.float32)
    for i in range(CH // 16):
        ones_v[pl.ds(i * 16, 16)] = z
    def _zero(j, _):
        pltpu.sync_copy(ones_v, acc_s.at[pl.ds(s * ROWS_PER_TILE + j * CH, CH)])
        return _
    lax.fori_loop(0, ROWS_PER_TILE // CH, _zero, None)
    o = jnp.ones((16,), jnp.float32)
    for i in range(CH // 16):
        ones_v[pl.ds(i * 16, 16)] = o
    pltpu.sync_copy(dst_hbm.at[pl.ds(wid * NCHUNK, NCHUNK)], idx_v)
    plsc.subcore_barrier()
    def _body(j, _):
        pltpu.sync_copy(ones_v, acc_s.at[idx_v.at[j]], add=True)
        return _
    lax.fori_loop(0, NCHUNK, _body, None)
    plsc.subcore_barrier()
    def _out(k, _):
        pltpu.sync_copy(acc_s.at[pl.ds(s * ROWS_PER_TILE + k * 128, 128)],
                        out_hbm.at[c, s * (ROWS_PER_TILE // 128) + k])
        return _
    lax.fori_loop(0, ROWS_PER_TILE // 128, _out, None)


@functools.partial(
    pl.kernel,
    out_type=jax.ShapeDtypeStruct((NC, NP, H), jnp.float32),
    mesh=_mesh,
    scratch_types=(
        [pltpu.VMEM((NCHUNK, CH), jnp.int32)] * 2
        + [pltpu.VMEM((CH, H), jnp.float32)] * NB
        + [pltpu.VMEM_SHARED((NP, H), jnp.float32)]
        + [pltpu.VMEM_SHARED((NP, H), jnp.float32)]
        + [pltpu.SemaphoreType.DMA] * (2 * NB)
    ),
    compiler_params=_sc_params,
)
def _scatter_kernel(g_hbm, src_hbm, dst_hbm, out_hbm, src_v, dst_v, *rest):
    rows = list(rest[:NB])
    acc_s = rest[NB]
    g_s = rest[NB + 1]
    sem_g = list(rest[NB + 2:NB + 2 + NB])
    sem_s = list(rest[NB + 2 + NB:])
    c = lax.axis_index("c")
    s = lax.axis_index("s")
    wid = c * NS + s
    z = jnp.zeros((16,), jnp.float32)
    def _zrow(j, _):
        rows[0][j, pl.ds(0, 16)] = z
        rows[0][j, pl.ds(16, 16)] = z
        return _
    lax.fori_loop(0, CH, _zrow, None)
    off = s * ROWS_PER_TILE
    for j in range(ROWS_PER_TILE // CH):
        pltpu.async_copy(rows[0], acc_s.at[pl.ds(off + j * CH, CH)], sem_s[j])
    pltpu.async_copy(src_hbm.at[pl.ds(wid * NCHUNK, NCHUNK)], src_v, sem_g[0])
    pltpu.async_copy(dst_hbm.at[pl.ds(wid * NCHUNK, NCHUNK)], dst_v, sem_g[1])
    pltpu.async_copy(g_hbm.at[pl.ds(off, ROWS_PER_TILE)],
                     g_s.at[pl.ds(off, ROWS_PER_TILE)], sem_g[2])
    for j in range(ROWS_PER_TILE // CH):
        pltpu.make_async_copy(rows[0], acc_s.at[pl.ds(off + j * CH, CH)],
                              sem_s[j]).wait()
    pltpu.make_async_copy(src_hbm.at[pl.ds(wid * NCHUNK, NCHUNK)], src_v,
                          sem_g[0]).wait()
    pltpu.make_async_copy(dst_hbm.at[pl.ds(wid * NCHUNK, NCHUNK)], dst_v,
                          sem_g[1]).wait()
    pltpu.make_async_copy(g_hbm.at[pl.ds(off, ROWS_PER_TILE)],
                          g_s.at[pl.ds(off, ROWS_PER_TILE)], sem_g[2]).wait()
    plsc.subcore_barrier()
    for k in range(IG):
        pltpu.async_copy(g_s.at[src_v.at[k]], rows[k], sem_g[k])
    def _body(i, _):
        for b in range(NB):
            j = NB * i + b
            pltpu.make_async_copy(g_s.at[src_v.at[j]], rows[b], sem_g[b]).wait()
            pltpu.async_copy(rows[b], acc_s.at[dst_v.at[j]], sem_s[b], add=True)
            bi = (b + IG) % NB
            @pl.when(j + IG < NCHUNK)
            def _issue():
                @pl.when(j + IG >= NB)
                def _drain():
                    pltpu.make_async_copy(
                        rows[bi], acc_s.at[dst_v.at[j]], sem_s[bi]).wait()
                pltpu.async_copy(g_s.at[src_v.at[j + IG]], rows[bi], sem_g[bi])
        return _
    lax.fori_loop(0, NCHUNK // NB, _body, None)
    for b in range(NB):
        pltpu.make_async_copy(rows[b], acc_s.at[dst_v.at[0]], sem_s[b]).wait()
    plsc.subcore_barrier()
    pltpu.sync_copy(acc_s.at[pl.ds(off, ROWS_PER_TILE)],
                    out_hbm.at[c, pl.ds(off, ROWS_PER_TILE)])


def _expand_rows(col_view):
    a_rows = lax.broadcasted_iota(jnp.int32, (BN, BR), 0) // 128
    a_cols = lax.broadcasted_iota(jnp.int32, (BN, BR), 1)
    sel = (a_rows == a_cols).astype(jnp.float32)
    o1 = jnp.dot(sel, col_view, preferred_element_type=jnp.float32)
    m_rows = lax.broadcasted_iota(jnp.int32, (BN, 128), 0) % 128
    m_cols = lax.broadcasted_iota(jnp.int32, (BN, 128), 1)
    msk = (m_rows == m_cols).astype(jnp.float32)
    return jnp.sum(o1 * msk, axis=1, keepdims=True)


def _tc1_body(x_ref, w_ref, d_ref, g_ref, dinv_ref):
    i = pl.program_id(0)
    dview = lax.rsqrt(d_ref[0] + d_ref[1] + 1.0)
    dinv = _expand_rows(dview)
    rows = i * BN + lax.broadcasted_iota(jnp.int32, (BN, 1), 0)
    h = jnp.dot(x_ref[...], w_ref[...], preferred_element_type=jnp.float32)
    valid = rows < N
    g_ref[...] = jnp.where(valid, h * dinv, 0.0)
    dinv_ref[...] = jnp.broadcast_to(jnp.where(valid, dinv, 0.0), (BN, H))


def _tc1(x, W1, deg):
    return pl.pallas_call(
        _tc1_body,
        out_shape=[jax.ShapeDtypeStruct((NP, H), jnp.float32),
                   jax.ShapeDtypeStruct((NP, H), jnp.float32)],
        grid=(GRID,),
        in_specs=[
            pl.BlockSpec((BN, D), lambda i: (i, 0)),
            pl.BlockSpec((D, H), lambda i: (0, 0)),
            pl.BlockSpec((NC, BR, 128), lambda i: (0, i, 0)),
        ],
        out_specs=[pl.BlockSpec((BN, H), lambda i: (i, 0)),
                   pl.BlockSpec((BN, H), lambda i: (i, 0))],
    )(x, W1, deg)


def _tc2_body(p_ref, g_ref, dinv_ref, b_ref, w_ref, out_ref):
    dinv = dinv_ref[...]
    pre = dinv * (p_ref[0] + p_ref[1] + g_ref[...]) + b_ref[...]
    h = jnp.maximum(pre, 0.0)
    out_ref[...] = jnp.dot(h, w_ref[...],
                           preferred_element_type=jnp.float32) * dinv


def _tc2(p, g1, dinv32, b1, W2):
    return pl.pallas_call(
        _tc2_body,
        out_shape=jax.ShapeDtypeStruct((NP, H), jnp.float32),
        grid=(GRID,),
        in_specs=[
            pl.BlockSpec((NC, BN, H), lambda i: (0, i, 0)),
            pl.BlockSpec((BN, H), lambda i: (i, 0)),
            pl.BlockSpec((BN, H), lambda i: (i, 0)),
            pl.BlockSpec((1, H), lambda i: (0, 0)),
            pl.BlockSpec((H, H), lambda i: (0, 0)),
        ],
        out_specs=pl.BlockSpec((BN, H), lambda i: (i, 0)),
    )(p, g1, dinv32, b1, W2)


def _tc3_body(q_ref, g_ref, dinv_ref, b_ref, batch_ref,
              wfc_ref, bfc_ref, wreg_ref, breg_ref, wcls_ref, bcls_ref,
              reg_ref, cls_ref, sums_ref, cnt_ref):
    i = pl.program_id(0)

    @pl.when(i == 0)
    def _init():
        sums_ref[...] = jnp.zeros((G, H), jnp.float32)
        cnt_ref[...] = jnp.zeros((G, 1), jnp.float32)

    pre = dinv_ref[...] * (q_ref[0] + q_ref[1] + g_ref[...]) + b_ref[...]
    h = jnp.maximum(pre, 0.0)
    bvals = batch_ref[...]
    gids = lax.broadcasted_iota(jnp.int32, (G, BN), 0)
    onehot_t = (gids == bvals).astype(jnp.float32)
    sums_ref[...] += jnp.dot(onehot_t, h, preferred_element_type=jnp.float32)
    cnt_ref[...] += jnp.sum(onehot_t, axis=1, keepdims=True)

    @pl.when(i == pl.num_programs(0) - 1)
    def _final():
        pooled = sums_ref[...] / jnp.maximum(cnt_ref[...], 1.0)
        sfc = jnp.maximum(
            jnp.dot(pooled, wfc_ref[...],
                    preferred_element_type=jnp.float32) + bfc_ref[...], 0.0)
        reg_ref[...] = jnp.dot(sfc, wreg_ref[...],
                               preferred_element_type=jnp.float32) + breg_ref[...]
        cls_ref[...] = jnp.dot(sfc, wcls_ref[...],
                               preferred_element_type=jnp.float32) + bcls_ref[...]


def _tc3(q, g2, dinv32, b2, batch2d, Wfc, bfc, Wreg, breg, Wcls, bcls):
    return pl.pallas_call(
        _tc3_body,
        out_shape=[jax.ShapeDtypeStruct((G, 2), jnp.float32),
                   jax.ShapeDtypeStruct((G, 2), jnp.float32)],
        grid=(GRID,),
        in_specs=[
            pl.BlockSpec((NC, BN, H), lambda i: (0, i, 0)),
            pl.BlockSpec((BN, H), lambda i: (i, 0)),
            pl.BlockSpec((BN, H), lambda i: (i, 0)),
            pl.BlockSpec((1, H), lambda i: (0, 0)),
            pl.BlockSpec((1, BN), lambda i: (0, i)),
            pl.BlockSpec((H, H), lambda i: (0, 0)),
            pl.BlockSpec((1, H), lambda i: (0, 0)),
            pl.BlockSpec((H, 2), lambda i: (0, 0)),
            pl.BlockSpec((1, 2), lambda i: (0, 0)),
            pl.BlockSpec((H, 2), lambda i: (0, 0)),
            pl.BlockSpec((1, 2), lambda i: (0, 0)),
        ],
        out_specs=[pl.BlockSpec((G, 2), lambda i: (0, 0)),
                   pl.BlockSpec((G, 2), lambda i: (0, 0))],
        scratch_shapes=[pltpu.VMEM((G, H), jnp.float32),
                        pltpu.VMEM((G, 1), jnp.float32)],
    )(q, g2, dinv32, b2, batch2d, Wfc, bfc, Wreg, breg, Wcls, bcls)


def kernel(x, edge_index, batch, W1, b1, W2, b2, Wfc, bfc, Wreg, breg,
           Wcls, bcls):
    pad_ids = (N + (jnp.arange(EP - E, dtype=jnp.int32) % (NP - N)))
    srcp = jnp.concatenate([edge_index[0], pad_ids]).reshape(EP // CH, CH)
    dstp = jnp.concatenate([edge_index[1], pad_ids]).reshape(EP // CH, CH)
    batchp = jnp.concatenate(
        [batch, jnp.full((NP - N,), G, jnp.int32)]).reshape(1, NP)

    deg = _deg_kernel(dstp)
    g1, dinv32 = _tc1(x, W1, deg)
    p = _scatter_kernel(g1, srcp, dstp)
    g2 = _tc2(p, g1, dinv32, b1.reshape(1, H), W2)
    q = _scatter_kernel(g2, srcp, dstp)
    reg, cls = _tc3(q, g2, dinv32, b2.reshape(1, H), batchp,
                    Wfc, bfc.reshape(1, H), Wreg, breg.reshape(1, 2),
                    Wcls, bcls.reshape(1, 2))
    return (reg, cls)


# speedup vs baseline: 44.8238x; 1.2698x over previous
import functools

import jax
import jax.numpy as jnp
from jax import lax
from jax.experimental import pallas as pl
from jax.experimental.pallas import tpu as pltpu
from jax.experimental.pallas import tpu_sc as plsc

N = 10000
E = 160000
D = 256
H = 32
G = 64

NC = 2
NS = 16
NP = 10240
NPV = NP // 4
ROWS_PER_TILE = NP // NS
CH = 128
EP = 172032
EDGES_PER_TILE = EP // (NC * NS)
NCHUNK = EDGES_PER_TILE // CH
NB = 7
IG = 4
BN = 2048
BNV = BN // 4
BR = BN // 128
GRID = NP // BN

_mesh = plsc.VectorSubcoreMesh(core_axis_name="c", subcore_axis_name="s")
_sc_params = pltpu.CompilerParams(use_tc_tiling_on_sc=False)


@functools.partial(
    pl.kernel,
    out_type=jax.ShapeDtypeStruct((NC, NP // 128, 128), jnp.float32),
    mesh=_mesh,
    scratch_types=[
        pltpu.VMEM((CH,), jnp.float32),
        pltpu.VMEM((NCHUNK, CH), jnp.int32),
        pltpu.VMEM_SHARED((NP,), jnp.float32),
    ],
    compiler_params=_sc_params,
)
def _deg_kernel(dst_hbm, out_hbm, ones_v, idx_v, acc_s):
    c = lax.axis_index("c")
    s = lax.axis_index("s")
    wid = c * NS + s
    z = jnp.zeros((16,), jnp.float32)
    for i in range(CH // 16):
        ones_v[pl.ds(i * 16, 16)] = z
    def _zero(j, _):
        pltpu.sync_copy(ones_v, acc_s.at[pl.ds(s * ROWS_PER_TILE + j * CH, CH)])
        return _
    lax.fori_loop(0, ROWS_PER_TILE // CH, _zero, None)
    o = jnp.ones((16,), jnp.float32)
    for i in range(CH // 16):
        ones_v[pl.ds(i * 16, 16)] = o
    pltpu.sync_copy(dst_hbm.at[pl.ds(wid * NCHUNK, NCHUNK)], idx_v)
    plsc.subcore_barrier()
    def _body(j, _):
        pltpu.sync_copy(ones_v, acc_s.at[idx_v.at[j]], add=True)
        return _
    lax.fori_loop(0, NCHUNK, _body, None)
    plsc.subcore_barrier()
    def _out(k, _):
        pltpu.sync_copy(acc_s.at[pl.ds(s * ROWS_PER_TILE + k * 128, 128)],
                        out_hbm.at[c, s * (ROWS_PER_TILE // 128) + k])
        return _
    lax.fori_loop(0, ROWS_PER_TILE // 128, _out, None)


@functools.partial(
    pl.kernel,
    out_type=jax.ShapeDtypeStruct((NC, NP, H), jnp.float32),
    mesh=_mesh,
    scratch_types=(
        [pltpu.VMEM((NCHUNK, CH), jnp.int32)] * 2
        + [pltpu.VMEM((CH, H), jnp.float32)] * NB
        + [pltpu.VMEM_SHARED((NP, H), jnp.float32)]
        + [pltpu.VMEM_SHARED((NP, H), jnp.float32)]
        + [pltpu.SemaphoreType.DMA] * (2 * NB)
    ),
    compiler_params=_sc_params,
)
def _scatter_kernel(g_hbm, src_hbm, dst_hbm, out_hbm, src_v, dst_v, *rest):
    rows = list(rest[:NB])
    acc_s = rest[NB]
    g_s = rest[NB + 1]
    sem_g = list(rest[NB + 2:NB + 2 + NB])
    sem_s = list(rest[NB + 2 + NB:])
    c = lax.axis_index("c")
    s = lax.axis_index("s")
    wid = c * NS + s
    z = jnp.zeros((16,), jnp.float32)
    def _zrow(j, _):
        rows[0][j, pl.ds(0, 16)] = z
        rows[0][j, pl.ds(16, 16)] = z
        return _
    lax.fori_loop(0, CH, _zrow, None)
    off = s * ROWS_PER_TILE
    for j in range(ROWS_PER_TILE // CH):
        pltpu.async_copy(rows[0], acc_s.at[pl.ds(off + j * CH, CH)], sem_s[j])
    pltpu.async_copy(src_hbm.at[pl.ds(wid * NCHUNK, NCHUNK)], src_v, sem_g[0])
    pltpu.async_copy(dst_hbm.at[pl.ds(wid * NCHUNK, NCHUNK)], dst_v, sem_g[1])
    pltpu.async_copy(g_hbm.at[pl.ds(off, ROWS_PER_TILE)],
                     g_s.at[pl.ds(off, ROWS_PER_TILE)], sem_g[2])
    for j in range(ROWS_PER_TILE // CH):
        pltpu.make_async_copy(rows[0], acc_s.at[pl.ds(off + j * CH, CH)],
                              sem_s[j]).wait()
    pltpu.make_async_copy(src_hbm.at[pl.ds(wid * NCHUNK, NCHUNK)], src_v,
                          sem_g[0]).wait()
    pltpu.make_async_copy(dst_hbm.at[pl.ds(wid * NCHUNK, NCHUNK)], dst_v,
                          sem_g[1]).wait()
    pltpu.make_async_copy(g_hbm.at[pl.ds(off, ROWS_PER_TILE)],
                          g_s.at[pl.ds(off, ROWS_PER_TILE)], sem_g[2]).wait()
    plsc.subcore_barrier()
    for k in range(IG):
        pltpu.async_copy(g_s.at[src_v.at[k]], rows[k], sem_g[k])
    def _body(i, _):
        for b in range(NB):
            j = NB * i + b
            pltpu.make_async_copy(g_s.at[src_v.at[j]], rows[b], sem_g[b]).wait()
            pltpu.async_copy(rows[b], acc_s.at[dst_v.at[j]], sem_s[b], add=True)
            bi = (b + IG) % NB
            @pl.when(j + IG < NCHUNK)
            def _issue():
                @pl.when(j + IG >= NB)
                def _drain():
                    pltpu.make_async_copy(
                        rows[bi], acc_s.at[dst_v.at[j]], sem_s[bi]).wait()
                pltpu.async_copy(g_s.at[src_v.at[j + IG]], rows[bi], sem_g[bi])
        return _
    lax.fori_loop(0, NCHUNK // NB, _body, None)
    for b in range(NB):
        pltpu.make_async_copy(rows[b], acc_s.at[dst_v.at[0]], sem_s[b]).wait()
    plsc.subcore_barrier()
    pltpu.sync_copy(acc_s.at[pl.ds(off, ROWS_PER_TILE)],
                    out_hbm.at[c, pl.ds(off, ROWS_PER_TILE)])


def _expand_rows(col_view):
    a_rows = lax.broadcasted_iota(jnp.int32, (BN, BR), 0) // 128
    a_cols = lax.broadcasted_iota(jnp.int32, (BN, BR), 1)
    sel = (a_rows == a_cols).astype(jnp.float32)
    o1 = jnp.dot(sel, col_view, preferred_element_type=jnp.float32)
    m_rows = lax.broadcasted_iota(jnp.int32, (BN, 128), 0) % 128
    m_cols = lax.broadcasted_iota(jnp.int32, (BN, 128), 1)
    msk = (m_rows == m_cols).astype(jnp.float32)
    return jnp.sum(o1 * msk, axis=1, keepdims=True)


def _expand_view(col_view):
    a_rows = lax.broadcasted_iota(jnp.int32, (BNV, BR), 0) // 32
    a_cols = lax.broadcasted_iota(jnp.int32, (BNV, BR), 1)
    sel = (a_rows == a_cols).astype(jnp.float32)
    o1 = jnp.dot(sel, col_view, preferred_element_type=jnp.float32)
    r_mod = lax.broadcasted_iota(jnp.int32, (BNV, 128), 0) % 32
    cols = lax.broadcasted_iota(jnp.int32, (BNV, 128), 1)
    parts = []
    for a in range(4):
        msk = (cols == 4 * r_mod + a).astype(jnp.float32)
        sa = jnp.sum(o1 * msk, axis=1, keepdims=True)
        parts.append(jnp.broadcast_to(sa, (BNV, 32)))
    return jnp.concatenate(parts, axis=1)


def _tc1_body(x_ref, w_ref, d_ref, g_ref, dinv_ref):
    i = pl.program_id(0)
    dview = lax.rsqrt(d_ref[0] + d_ref[1])
    dinv = _expand_rows(dview)
    rows = i * BN + lax.broadcasted_iota(jnp.int32, (BN, 1), 0)
    h = jnp.dot(x_ref[...], w_ref[...], preferred_element_type=jnp.float32)
    g_ref[...] = jnp.where(rows < N, h * dinv, 0.0)
    dinv_ref[...] = _expand_view(dview)


def _tc1(x, W1, deg):
    return pl.pallas_call(
        _tc1_body,
        out_shape=[jax.ShapeDtypeStruct((NP, H), jnp.float32),
                   jax.ShapeDtypeStruct((NPV, 128), jnp.float32)],
        grid=(GRID,),
        in_specs=[
            pl.BlockSpec((BN, D), lambda i: (i, 0)),
            pl.BlockSpec((D, H), lambda i: (0, 0)),
            pl.BlockSpec((NC, BR, 128), lambda i: (0, i, 0)),
        ],
        out_specs=[pl.BlockSpec((BN, H), lambda i: (i, 0)),
                   pl.BlockSpec((BNV, 128), lambda i: (i, 0))],
    )(x, W1, deg)


def _tc2_body(p_ref, dinv_ref, b_ref, w_ref, out_ref):
    dinv = dinv_ref[...]
    pre = dinv * (p_ref[0] + p_ref[1]) + b_ref[...]
    h = jnp.maximum(pre, 0.0)
    out_ref[...] = jnp.dot(h, w_ref[...],
                           preferred_element_type=jnp.float32) * dinv


def _tc2(p, dinv_v, b1t, W2k):
    return pl.pallas_call(
        _tc2_body,
        out_shape=jax.ShapeDtypeStruct((NPV, 128), jnp.float32),
        grid=(GRID,),
        in_specs=[
            pl.BlockSpec((NC, BNV, 128), lambda i: (0, i, 0)),
            pl.BlockSpec((BNV, 128), lambda i: (i, 0)),
            pl.BlockSpec((1, 128), lambda i: (0, 0)),
            pl.BlockSpec((128, 128), lambda i: (0, 0)),
        ],
        out_specs=pl.BlockSpec((BNV, 128), lambda i: (i, 0)),
    )(p, dinv_v, b1t, W2k)


def _tc3_body(q_ref, dinv_ref, b_ref, batch_ref,
              wfc_ref, bfc_ref, wreg_ref, breg_ref, wcls_ref, bcls_ref,
              reg_ref, cls_ref, sums_ref, cnt_ref):
    i = pl.program_id(0)

    @pl.when(i == 0)
    def _init():
        sums_ref[...] = jnp.zeros((G, H), jnp.float32)
        cnt_ref[...] = jnp.zeros((G, 1), jnp.float32)

    pre = dinv_ref[...] * (q_ref[0] + q_ref[1]) + b_ref[...]
    h = jnp.maximum(pre, 0.0)
    b4 = batch_ref[...]
    gids = lax.broadcasted_iota(jnp.int32, (G, BNV), 0)
    for a in range(4):
        onehot_t = (gids == b4[a:a + 1, :]).astype(jnp.float32)
        sa = jnp.dot(onehot_t, h, preferred_element_type=jnp.float32)
        sums_ref[...] += sa[:, 32 * a:32 * a + 32]
        cnt_ref[...] += jnp.sum(onehot_t, axis=1, keepdims=True)

    @pl.when(i == pl.num_programs(0) - 1)
    def _final():
        pooled = sums_ref[...] / jnp.maximum(cnt_ref[...], 1.0)
        sfc = jnp.maximum(
            jnp.dot(pooled, wfc_ref[...],
                    preferred_element_type=jnp.float32) + bfc_ref[...], 0.0)
        reg_ref[...] = jnp.dot(sfc, wreg_ref[...],
                               preferred_element_type=jnp.float32) + breg_ref[...]
        cls_ref[...] = jnp.dot(sfc, wcls_ref[...],
                               preferred_element_type=jnp.float32) + bcls_ref[...]


def _tc3(q, dinv_v, b2t, batch4, Wfc, bfc, Wreg, breg, Wcls, bcls):
    return pl.pallas_call(
        _tc3_body,
        out_shape=[jax.ShapeDtypeStruct((G, 2), jnp.float32),
                   jax.ShapeDtypeStruct((G, 2), jnp.float32)],
        grid=(GRID,),
        in_specs=[
            pl.BlockSpec((NC, BNV, 128), lambda i: (0, i, 0)),
            pl.BlockSpec((BNV, 128), lambda i: (i, 0)),
            pl.BlockSpec((1, 128), lambda i: (0, 0)),
            pl.BlockSpec((4, BNV), lambda i: (0, i)),
            pl.BlockSpec((H, H), lambda i: (0, 0)),
            pl.BlockSpec((1, H), lambda i: (0, 0)),
            pl.BlockSpec((H, 2), lambda i: (0, 0)),
            pl.BlockSpec((1, 2), lambda i: (0, 0)),
            pl.BlockSpec((H, 2), lambda i: (0, 0)),
            pl.BlockSpec((1, 2), lambda i: (0, 0)),
        ],
        out_specs=[pl.BlockSpec((G, 2), lambda i: (0, 0)),
                   pl.BlockSpec((G, 2), lambda i: (0, 0))],
        scratch_shapes=[pltpu.VMEM((G, H), jnp.float32),
                        pltpu.VMEM((G, 1), jnp.float32)],
    )(q, dinv_v, b2t, batch4, Wfc, bfc, Wreg, breg, Wcls, bcls)


def kernel(x, edge_index, batch, W1, b1, W2, b2, Wfc, bfc, Wreg, breg,
           Wcls, bcls):
    loops = jnp.arange(NP, dtype=jnp.int32)
    pad_ids = (N + (jnp.arange(EP - E - NP, dtype=jnp.int32) % (NP - N)))
    srcp = jnp.concatenate([edge_index[0], loops, pad_ids]).reshape(
        EP // CH, CH)
    dstp = jnp.concatenate([edge_index[1], loops, pad_ids]).reshape(
        EP // CH, CH)
    batchp = jnp.concatenate([batch, jnp.full((NP - N,), G, jnp.int32)])
    batch4 = batchp.reshape(NPV, 4).T
    eye4 = jnp.eye(4, dtype=jnp.float32)
    W2k = jnp.kron(eye4, W2)
    b1t = jnp.tile(b1.reshape(1, H), (1, 4))
    b2t = jnp.tile(b2.reshape(1, H), (1, 4))

    deg = _deg_kernel(dstp)
    g1, dinv_v = _tc1(x, W1, deg)
    p = _scatter_kernel(g1, srcp, dstp).reshape(NC, NPV, 128)
    g2v = _tc2(p, dinv_v, b1t, W2k)
    q = _scatter_kernel(g2v.reshape(NP, H), srcp, dstp).reshape(NC, NPV, 128)
    reg, cls = _tc3(q, dinv_v, b2t, batch4,
                    Wfc, bfc.reshape(1, H), Wreg, breg.reshape(1, 2),
                    Wcls, bcls.reshape(1, 2))
    return (reg, cls)
